# baseline (speedup 1.0000x reference)
import functools

import jax
import jax.numpy as jnp
from jax import lax
from jax.experimental import pallas as pl
from jax.experimental.pallas import tpu as pltpu
from jax.experimental.pallas import tpu_sc as plsc

_V = 1000000
_D = 32
_B = 16384
_CHUNK = 128
_L = 16

_info = plsc.get_sparse_core_info()
_NC, _NS = _info.num_cores, _info.num_subcores
_NW = _NC * _NS
_B_PER_W = _B // _NW
_N_CHUNK = _B_PER_W // _CHUNK

_mesh = plsc.VectorSubcoreMesh(core_axis_name="c", subcore_axis_name="s")


@functools.partial(
    pl.kernel,
    mesh=_mesh,
    compiler_params=pltpu.CompilerParams(use_tc_tiling_on_sc=False),
    out_type=jax.ShapeDtypeStruct((_D, _B, 1), jnp.float32),
    scratch_types=[
        pltpu.VMEM((_N_CHUNK, _CHUNK), jnp.int32),
        pltpu.VMEM((_N_CHUNK, _CHUNK), jnp.int32),
        pltpu.VMEM((_D // 2, _B_PER_W, 1), jnp.float32),
        pltpu.SemaphoreType.DMA,
    ],
)
def _embed(idx_hbm, table_hbm, out_hbm, p_v, adr_v, out_v, sem):
    wid = lax.axis_index("s") * _NC + lax.axis_index("c")
    base = wid * _N_CHUNK
    pltpu.sync_copy(idx_hbm.at[pl.ds(base, _N_CHUNK)], p_v)
    tab_flat = table_hbm

    for k in range(_B_PER_W // _L):
        r, c = k // (_CHUNK // _L), (k % (_CHUNK // _L)) * _L
        p_v[r, pl.ds(c, _L)] = lax.shift_left(p_v[r, pl.ds(c, _L)], 5)

    for h in range(2):

        def per_dim(d, _):
            for k in range(_B_PER_W // _L):
                r, c = k // (_CHUNK // _L), (k % (_CHUNK // _L)) * _L
                adr_v[r, pl.ds(c, _L)] = p_v[r, pl.ds(c, _L)] + (h * (_D // 2) + d)
            copies = [
                pltpu.async_copy(
                    tab_flat.at[adr_v.at[j]],
                    out_v.at[d, pl.ds(j * _CHUNK, _CHUNK)],
                    sem,
                )
                for j in range(_N_CHUNK)
            ]
            for cp in copies:
                cp.wait()
            return 0

        lax.fori_loop(0, _D // 2, per_dim, 0)
        pltpu.sync_copy(
            out_v,
            out_hbm.at[pl.ds(h * (_D // 2), _D // 2), pl.ds(wid * _B_PER_W, _B_PER_W)],
        )


def kernel(x, weight):
    idx = x.astype(jnp.int32).reshape(_B // _CHUNK, _CHUNK)
    return _embed(idx, weight.reshape(_V * _D, 1)).reshape(_D, _B).T


# speedup vs baseline: 71.9853x; 71.9853x over previous
import functools

import jax
import jax.numpy as jnp
from jax import lax
from jax.experimental import pallas as pl
from jax.experimental.pallas import tpu as pltpu
from jax.experimental.pallas import tpu_sc as plsc

_D = 32
_B = 16384
_CHUNK = 128
_L = 16

_info = plsc.get_sparse_core_info()
_NC, _NS = _info.num_cores, _info.num_subcores
_NW = _NC * _NS
_B_PER_W = _B // _NW
_N_CHUNK = _B_PER_W // _CHUNK
_N_GROUP = _B_PER_W // _L

_mesh = plsc.VectorSubcoreMesh(core_axis_name="c", subcore_axis_name="s")


def _perm(v, p):
    dnums = lax.GatherDimensionNumbers(
        offset_dims=(), collapsed_slice_dims=(0,), start_index_map=(0,)
    )
    return lax.gather(
        v, p[:, None], dnums, (1,),
        mode=lax.GatherScatterMode.PROMISE_IN_BOUNDS,
    )


def _transpose16(vs):
    iota = lax.iota(jnp.int32, _L)
    for s in (1, 2, 4, 8):
        keep = (iota & s) == 0
        shl_p = (iota - s) % _L
        shr_p = (iota + s) % _L
        nxt = list(vs)
        for r in range(_L):
            if r & s == 0:
                p = r | s
                a, b = vs[r], vs[p]
                nxt[r] = jnp.where(keep, a, _perm(b, shl_p))
                nxt[p] = jnp.where(keep, _perm(a, shr_p), b)
        vs = nxt
    return vs


@functools.partial(
    pl.kernel,
    mesh=_mesh,
    out_type=jax.ShapeDtypeStruct((_D, _B), jnp.float32),
    scratch_types=[
        pltpu.VMEM((_N_CHUNK, _CHUNK), jnp.int32),
        pltpu.VMEM((_N_CHUNK, _CHUNK), jnp.int32),
        pltpu.VMEM((_B_PER_W,), jnp.int32),
        pltpu.VMEM((_B_PER_W, 4 * _D), jnp.float32),
        pltpu.VMEM((_D, _B_PER_W), jnp.float32),
        pltpu.SemaphoreType.DMA,
    ],
)
def _embed(idx_hbm, table_hbm, out_hbm, idx_v, idx4_v, cb_v, buf_v, out_v, sem):
    wid = lax.axis_index("s") * _NC + lax.axis_index("c")
    base = wid * _N_CHUNK
    pltpu.sync_copy(idx_hbm.at[pl.ds(base, _N_CHUNK)], idx_v)

    for k in range(_B_PER_W // _L):
        r, c = k // (_CHUNK // _L), (k % (_CHUNK // _L)) * _L
        t = idx_v[r, pl.ds(c, _L)]
        idx4_v[r, pl.ds(c, _L)] = lax.shift_right_logical(t, 2)
        cb_v[pl.ds(k * _L, _L)] = lax.shift_left(t & 3, 5)

    copies = [
        pltpu.async_copy(
            table_hbm.at[idx4_v.at[j]], buf_v.at[pl.ds(j * _CHUNK, _CHUNK)], sem
        )
        for j in range(_N_CHUNK)
    ]
    for cp in copies:
        cp.wait()

    def select(g, _):
        g16 = g * _L
        cbv = cb_v[pl.ds(g16, _L)]
        for cg in range(_D // _L):
            vs = []
            for k in range(_L):
                cb = cbv[k]
                vs.append(buf_v[g16 + k, pl.ds(cb + cg * _L, _L)])
            ws = _transpose16(vs)
            for c in range(_L):
                out_v[cg * _L + c, pl.ds(g16, _L)] = ws[c]
        return 0

    lax.fori_loop(0, _N_GROUP, select, 0)
    pltpu.sync_copy(out_v, out_hbm.at[:, pl.ds(wid * _B_PER_W, _B_PER_W)])


def kernel(x, weight):
    idx = x.astype(jnp.int32).reshape(_B // _CHUNK, _CHUNK)
    table = weight.reshape(250000, 4 * _D)
    return _embed(idx, table).T
